# baseline (speedup 1.0000x reference)
import jax
import jax.numpy as jnp
from jax import lax
from jax.experimental import pallas as pl
from jax.experimental.pallas import tpu as pltpu
from jax.experimental.pallas import tpu_sc as plsc

_N = 10000
_E = 320000
_ANN = 64
_OUT = 128
_STEPS = 5
_CLS = 16
_GRAPHS = 256
_RB = 1000


def _matmul_t(a, w):
    return lax.dot_general(a, w, (((1,), (1,)), ((), ())),
                           preferred_element_type=jnp.float32)


def _edge_lin_body(feat_ref, wlin_ref, blin_ref, m_ref):
    m_ref[...] = _matmul_t(feat_ref[...], wlin_ref[...]) + blin_ref[...]


def _edge_lin(feat, W_lin, b_lin2):
    return pl.pallas_call(
        _edge_lin_body,
        grid=(_N // _RB,),
        in_specs=[
            pl.BlockSpec((_RB, _OUT), lambda i: (i, 0)),
            pl.BlockSpec((_OUT, _OUT), lambda i: (0, 0)),
            pl.BlockSpec((1, _OUT), lambda i: (0, 0)),
        ],
        out_specs=pl.BlockSpec((_RB, _OUT), lambda i: (i, 0)),
        out_shape=jax.ShapeDtypeStruct((_N, _OUT), jnp.float32),
    )(feat, W_lin, b_lin2)


def _whh_body(h_ref, whh_ref, bhh_ref, gh_ref):
    gh_ref[...] = _matmul_t(h_ref[...], whh_ref[...]) + bhh_ref[...]


def _whh_mm(h, W_hh, b_hh2):
    return pl.pallas_call(
        _whh_body,
        grid=(_N // _RB,),
        in_specs=[
            pl.BlockSpec((_RB, _OUT), lambda i: (i, 0)),
            pl.BlockSpec((3 * _OUT, _OUT), lambda i: (0, 0)),
            pl.BlockSpec((1, 3 * _OUT), lambda i: (0, 0)),
        ],
        out_specs=pl.BlockSpec((_RB, 3 * _OUT), lambda i: (i, 0)),
        out_shape=jax.ShapeDtypeStruct((_N, 3 * _OUT), jnp.float32),
    )(h, W_hh, b_hh2)


def _gru_body(pp_ref, h_ref, gh_ref, wlin_ref, blin_ref, wih_ref,
              bih_ref, h_out_ref, m_out_ref):
    a = pp_ref[0] + pp_ref[1]
    h = h_ref[...]
    gi = _matmul_t(a, wih_ref[...]) + bih_ref[...]
    gh = gh_ref[...]
    r = jax.nn.sigmoid(gi[:, :_OUT] + gh[:, :_OUT])
    z = jax.nn.sigmoid(gi[:, _OUT:2 * _OUT] + gh[:, _OUT:2 * _OUT])
    n = jnp.tanh(gi[:, 2 * _OUT:] + r * gh[:, 2 * _OUT:])
    h_new = (1.0 - z) * n + z * h
    h_out_ref[...] = h_new
    m_out_ref[...] = _matmul_t(h_new, wlin_ref[...]) + blin_ref[...]


def _gru_step(p_parts, h, gh, W_lin, b_lin2, W_ih, b_ih2):
    return pl.pallas_call(
        _gru_body,
        grid=(_N // _RB,),
        in_specs=[
            pl.BlockSpec((2, _RB, _OUT), lambda i: (0, i, 0)),
            pl.BlockSpec((_RB, _OUT), lambda i: (i, 0)),
            pl.BlockSpec((_RB, 3 * _OUT), lambda i: (i, 0)),
            pl.BlockSpec((_OUT, _OUT), lambda i: (0, 0)),
            pl.BlockSpec((1, _OUT), lambda i: (0, 0)),
            pl.BlockSpec((3 * _OUT, _OUT), lambda i: (0, 0)),
            pl.BlockSpec((1, 3 * _OUT), lambda i: (0, 0)),
        ],
        out_specs=[
            pl.BlockSpec((_RB, _OUT), lambda i: (i, 0)),
            pl.BlockSpec((_RB, _OUT), lambda i: (i, 0)),
        ],
        out_shape=[
            jax.ShapeDtypeStruct((_N, _OUT), jnp.float32),
            jax.ShapeDtypeStruct((_N, _OUT), jnp.float32),
        ],
    )(p_parts, h, gh, W_lin, b_lin2, W_ih, b_ih2)


def _readout_body(h_ref, x_ref, gid_ref, lab_ref, wout_ref, bout_ref,
                  loss_ref, preds_ref, acc_ref):
    i = pl.program_id(0)

    @pl.when(i == 0)
    def _():
        acc_ref[...] = jnp.zeros_like(acc_ref)

    out = jnp.concatenate([h_ref[...], x_ref[...]], axis=1)
    mx = jnp.max(out, axis=1, keepdims=True)
    e = jnp.exp(out - mx)
    gate = e / jnp.sum(e, axis=1, keepdims=True)
    r = out * gate
    gids = gid_ref[0, 0, :]
    onehot = (gids[None, :] == lax.broadcasted_iota(jnp.int32, (_GRAPHS, _RB), 0)
              ).astype(jnp.float32)
    acc_ref[...] += lax.dot_general(onehot, r, (((1,), (0,)), ((), ())),
                                    preferred_element_type=jnp.float32)

    @pl.when(i == (_N // _RB) - 1)
    def _():
        logits = lax.dot_general(acc_ref[...], wout_ref[...],
                                 (((1,), (0,)), ((), ())),
                                 preferred_element_type=jnp.float32) + bout_ref[...]
        amax = jnp.max(logits, axis=1, keepdims=True)
        cls_iota = lax.broadcasted_iota(jnp.int32, (_GRAPHS, _CLS), 1)
        preds = jnp.min(jnp.where(logits == amax, cls_iota, _CLS), axis=1)
        preds_ref[...] = preds[None, :]
        lse = amax + jnp.log(jnp.sum(jnp.exp(logits - amax), axis=1, keepdims=True))
        logp = logits - lse
        lab = lab_ref[0, :]
        lab_oh = (lab[:, None] == cls_iota).astype(jnp.float32)
        loss_ref[...] = -jnp.sum(logp * lab_oh, keepdims=True) / _GRAPHS


def _readout(h, x, gid3, lab2, W_out, b_out2):
    return pl.pallas_call(
        _readout_body,
        grid=(_N // _RB,),
        in_specs=[
            pl.BlockSpec((_RB, _OUT), lambda i: (i, 0)),
            pl.BlockSpec((_RB, _ANN), lambda i: (i, 0)),
            pl.BlockSpec((1, 1, _RB), lambda i: (i, 0, 0)),
            pl.BlockSpec((1, _GRAPHS), lambda i: (0, 0)),
            pl.BlockSpec((_ANN + _OUT, _CLS), lambda i: (0, 0)),
            pl.BlockSpec((1, _CLS), lambda i: (0, 0)),
        ],
        out_specs=[
            pl.BlockSpec((1, 1), lambda i: (0, 0)),
            pl.BlockSpec((1, _GRAPHS), lambda i: (0, 0)),
        ],
        out_shape=[
            jax.ShapeDtypeStruct((1, 1), jnp.float32),
            jax.ShapeDtypeStruct((1, _GRAPHS), jnp.int32),
        ],
        scratch_shapes=[pltpu.VMEM((_GRAPHS, _ANN + _OUT), jnp.float32)],
    )(h, x, gid3, lab2, W_out, b_out2)


_NS = 16
_NC = 2
_NW = _NC * _NS
_CH = 125
_EPT = _E // _NW
_CB = 16
_NB = _EPT // (_CB * _CH)
_NPAIR = _CB // 2
_RTB = 624
_ZB = 48
_TAIL = _N - _NS * _RTB


def _agg_body(m_hbm, src_hbm, dst_hbm, out_hbm, src_v, dst_v, rb0, rb1,
              gs0, gs1, ss0, ss1, acc_sh):
    cid = lax.axis_index("c")
    sid = lax.axis_index("s")
    wid = sid * _NC + cid
    bufs = [rb0, rb1]
    gsems = [gs0, gs1]
    ssems = [ss0, ss1]

    def zrow(r, carry):
        for c16 in range(_OUT // 16):
            rb0[r, pl.ds(c16 * 16, 16)] = jnp.zeros((16,), jnp.float32)
        return carry
    lax.fori_loop(0, _ZB, zrow, 0)
    for k in range(_RTB // _ZB):
        pltpu.sync_copy(rb0.at[pl.ds(0, _ZB)],
                        acc_sh.at[pl.ds(sid * _RTB + k * _ZB, _ZB)])

    @pl.when(sid == _NS - 1)
    def _():
        pltpu.sync_copy(rb0.at[pl.ds(0, _TAIL)],
                        acc_sh.at[pl.ds(_NS * _RTB, _TAIL)])

    plsc.subcore_barrier()

    def _wait_gather(c, j):
        pltpu.make_async_copy(m_hbm.at[src_v.at[c]], bufs[j], gsems[j]).wait()

    def _wait_scatter(j):
        pltpu.make_async_copy(bufs[j], acc_sh.at[dst_v.at[0]], ssems[j]).wait()

    def block(b, carry):
        pltpu.sync_copy(src_hbm.at[wid, b], src_v)
        pltpu.sync_copy(dst_hbm.at[wid, b], dst_v)
        pltpu.async_copy(m_hbm.at[src_v.at[0]], bufs[0], gsems[0])
        pltpu.async_copy(m_hbm.at[src_v.at[1]], bufs[1], gsems[1])

        def pair(i, carry2):
            c0 = 2 * i
            _wait_gather(c0, 0)
            pltpu.sync_copy(bufs[0], acc_sh.at[dst_v.at[c0]], add=True)

            @pl.when(i < _NPAIR - 1)
            def _():
                pltpu.async_copy(m_hbm.at[src_v.at[c0 + 2]], bufs[0],
                                 gsems[0])
            _wait_gather(c0 + 1, 1)
            pltpu.sync_copy(bufs[1], acc_sh.at[dst_v.at[c0 + 1]], add=True)

            @pl.when(i < _NPAIR - 1)
            def _():
                pltpu.async_copy(m_hbm.at[src_v.at[c0 + 3]], bufs[1],
                                 gsems[1])
            return carry2
        return lax.fori_loop(0, _NPAIR, pair, carry)
    lax.fori_loop(0, _NB, block, 0)

    plsc.subcore_barrier()
    pltpu.sync_copy(acc_sh.at[pl.ds(sid * _RTB, _RTB)],
                    out_hbm.at[cid, pl.ds(sid * _RTB, _RTB)])

    @pl.when(sid == _NS - 1)
    def _():
        pltpu.sync_copy(acc_sh.at[pl.ds(_NS * _RTB, _TAIL)],
                        out_hbm.at[cid, pl.ds(_NS * _RTB, _TAIL)])


_agg_call = pl.kernel(
    _agg_body,
    out_type=jax.ShapeDtypeStruct((_NC, _N, _OUT), jnp.float32),
    mesh=plsc.VectorSubcoreMesh(core_axis_name="c", subcore_axis_name="s"),
    scratch_types=(
        [pltpu.VMEM((_CB, _CH), jnp.int32)] * 2
        + [pltpu.VMEM((_CH, _OUT), jnp.float32)] * 2
        + [pltpu.SemaphoreType.DMA] * 4
        + [pltpu.VMEM_SHARED((_N, _OUT), jnp.float32)]
    ),
)


def kernel(x, edge_index, graph_ids, labels, W_lin, b_lin, W_ih, W_hh,
           b_ih, b_hh, W_gate, b_gate, W_out, b_out):
    feat = jnp.concatenate(
        [x, jnp.zeros((_N, _OUT - _ANN), jnp.float32)], axis=1)
    src_rs = edge_index[0].reshape(_NW, _NB, _CB, _CH)
    dst_rs = edge_index[1].reshape(_NW, _NB, _CB, _CH)
    b_lin2 = b_lin[None, :]
    b_ih2 = b_ih[None, :]
    b_hh2 = b_hh[None, :]
    b_out2 = b_out[None, :]
    gid3 = graph_ids.reshape(_N // _RB, 1, _RB)
    lab2 = labels[None, :]

    h = feat
    m = _edge_lin(feat, W_lin, b_lin2)
    gh = _whh_mm(h, W_hh, b_hh2)
    for s in range(_STEPS):
        p_parts = _agg_call(m, src_rs, dst_rs)
        h, m = _gru_step(p_parts, h, gh, W_lin, b_lin2, W_ih, b_ih2)
        if s < _STEPS - 1:
            gh = _whh_mm(h, W_hh, b_hh2)
    loss2, preds2 = _readout(h, x, gid3, lab2, W_out, b_out2)
    return (loss2.reshape(()), preds2.reshape(_GRAPHS))


# speedup vs baseline: 1.0838x; 1.0838x over previous
import jax
import jax.numpy as jnp
from jax import lax
from jax.experimental import pallas as pl
from jax.experimental.pallas import tpu as pltpu
from jax.experimental.pallas import tpu_sc as plsc

_N = 10000
_E = 320000
_ANN = 64
_OUT = 128
_STEPS = 5
_CLS = 16
_GRAPHS = 256
_RB = 1000


def _matmul_t(a, w):
    return lax.dot_general(a, w, (((1,), (1,)), ((), ())),
                           preferred_element_type=jnp.float32)


def _edge_lin_body(feat_ref, wlin_ref, blin_ref, m_ref):
    m_ref[...] = _matmul_t(feat_ref[...], wlin_ref[...]) + blin_ref[...]


def _edge_lin(feat, W_lin, b_lin2):
    return pl.pallas_call(
        _edge_lin_body,
        grid=(_N // _RB,),
        in_specs=[
            pl.BlockSpec((_RB, _OUT), lambda i: (i, 0)),
            pl.BlockSpec((_OUT, _OUT), lambda i: (0, 0)),
            pl.BlockSpec((1, _OUT), lambda i: (0, 0)),
        ],
        out_specs=pl.BlockSpec((_RB, _OUT), lambda i: (i, 0)),
        out_shape=jax.ShapeDtypeStruct((_N, _OUT), jnp.float32),
    )(feat, W_lin, b_lin2)


def _gru_body(pp_ref, h_ref, wlin_ref, blin_ref, wih_ref, whh_ref,
              bih_ref, bhh_ref, h_out_ref, m_out_ref):
    a = pp_ref[0] + pp_ref[1]
    h = h_ref[...]
    gi = _matmul_t(a, wih_ref[...]) + bih_ref[...]
    gh = _matmul_t(h, whh_ref[...]) + bhh_ref[...]
    r = jax.nn.sigmoid(gi[:, :_OUT] + gh[:, :_OUT])
    z = jax.nn.sigmoid(gi[:, _OUT:2 * _OUT] + gh[:, _OUT:2 * _OUT])
    n = jnp.tanh(gi[:, 2 * _OUT:] + r * gh[:, 2 * _OUT:])
    h_new = (1.0 - z) * n + z * h
    h_out_ref[...] = h_new
    m_out_ref[...] = _matmul_t(h_new, wlin_ref[...]) + blin_ref[...]


def _gru_step(p_parts, h, W_lin, b_lin2, W_ih, W_hh, b_ih2, b_hh2):
    return pl.pallas_call(
        _gru_body,
        grid=(_N // _RB,),
        in_specs=[
            pl.BlockSpec((2, _RB, _OUT), lambda i: (0, i, 0)),
            pl.BlockSpec((_RB, _OUT), lambda i: (i, 0)),
            pl.BlockSpec((_OUT, _OUT), lambda i: (0, 0)),
            pl.BlockSpec((1, _OUT), lambda i: (0, 0)),
            pl.BlockSpec((3 * _OUT, _OUT), lambda i: (0, 0)),
            pl.BlockSpec((3 * _OUT, _OUT), lambda i: (0, 0)),
            pl.BlockSpec((1, 3 * _OUT), lambda i: (0, 0)),
            pl.BlockSpec((1, 3 * _OUT), lambda i: (0, 0)),
        ],
        out_specs=[
            pl.BlockSpec((_RB, _OUT), lambda i: (i, 0)),
            pl.BlockSpec((_RB, _OUT), lambda i: (i, 0)),
        ],
        out_shape=[
            jax.ShapeDtypeStruct((_N, _OUT), jnp.float32),
            jax.ShapeDtypeStruct((_N, _OUT), jnp.float32),
        ],
    )(p_parts, h, W_lin, b_lin2, W_ih, W_hh, b_ih2, b_hh2)


def _readout_body(h_ref, x_ref, gid_ref, lab_ref, wout_ref, bout_ref,
                  loss_ref, preds_ref, acc_ref):
    i = pl.program_id(0)

    @pl.when(i == 0)
    def _():
        acc_ref[...] = jnp.zeros_like(acc_ref)

    out = jnp.concatenate([h_ref[...], x_ref[...]], axis=1)
    mx = jnp.max(out, axis=1, keepdims=True)
    e = jnp.exp(out - mx)
    gate = e / jnp.sum(e, axis=1, keepdims=True)
    r = out * gate
    gids = gid_ref[0, 0, :]
    onehot = (gids[None, :] == lax.broadcasted_iota(jnp.int32, (_GRAPHS, _RB), 0)
              ).astype(jnp.float32)
    acc_ref[...] += lax.dot_general(onehot, r, (((1,), (0,)), ((), ())),
                                    preferred_element_type=jnp.float32)

    @pl.when(i == (_N // _RB) - 1)
    def _():
        logits = lax.dot_general(acc_ref[...], wout_ref[...],
                                 (((1,), (0,)), ((), ())),
                                 preferred_element_type=jnp.float32) + bout_ref[...]
        amax = jnp.max(logits, axis=1, keepdims=True)
        cls_iota = lax.broadcasted_iota(jnp.int32, (_GRAPHS, _CLS), 1)
        preds = jnp.min(jnp.where(logits == amax, cls_iota, _CLS), axis=1)
        preds_ref[...] = preds[None, :]
        lse = amax + jnp.log(jnp.sum(jnp.exp(logits - amax), axis=1, keepdims=True))
        logp = logits - lse
        lab = lab_ref[0, :]
        lab_oh = (lab[:, None] == cls_iota).astype(jnp.float32)
        loss_ref[...] = -jnp.sum(logp * lab_oh, keepdims=True) / _GRAPHS


def _readout(h, x, gid3, lab2, W_out, b_out2):
    return pl.pallas_call(
        _readout_body,
        grid=(_N // _RB,),
        in_specs=[
            pl.BlockSpec((_RB, _OUT), lambda i: (i, 0)),
            pl.BlockSpec((_RB, _ANN), lambda i: (i, 0)),
            pl.BlockSpec((1, 1, _RB), lambda i: (i, 0, 0)),
            pl.BlockSpec((1, _GRAPHS), lambda i: (0, 0)),
            pl.BlockSpec((_ANN + _OUT, _CLS), lambda i: (0, 0)),
            pl.BlockSpec((1, _CLS), lambda i: (0, 0)),
        ],
        out_specs=[
            pl.BlockSpec((1, 1), lambda i: (0, 0)),
            pl.BlockSpec((1, _GRAPHS), lambda i: (0, 0)),
        ],
        out_shape=[
            jax.ShapeDtypeStruct((1, 1), jnp.float32),
            jax.ShapeDtypeStruct((1, _GRAPHS), jnp.int32),
        ],
        scratch_shapes=[pltpu.VMEM((_GRAPHS, _ANN + _OUT), jnp.float32)],
    )(h, x, gid3, lab2, W_out, b_out2)


_NS = 16
_NC = 2
_NW = _NC * _NS
_CH = 125
_EPT = _E // _NW
_CB = 40
_NB = _EPT // (_CB * _CH)
_NPAIR = _CB // 2
_RTB = 624
_ZB = 48
_TAIL = _N - _NS * _RTB


def _agg_body(m_hbm, src_hbm, dst_hbm, out_hbm, src_v, dst_v, rb0, rb1,
              gs0, gs1, ss0, ss1, acc_sh):
    cid = lax.axis_index("c")
    sid = lax.axis_index("s")
    wid = sid * _NC + cid
    bufs = [rb0, rb1]
    gsems = [gs0, gs1]
    ssems = [ss0, ss1]

    def zrow(r, carry):
        for c16 in range(_OUT // 16):
            rb0[r, pl.ds(c16 * 16, 16)] = jnp.zeros((16,), jnp.float32)
        return carry
    lax.fori_loop(0, _ZB, zrow, 0)
    for k in range(_RTB // _ZB):
        pltpu.sync_copy(rb0.at[pl.ds(0, _ZB)],
                        acc_sh.at[pl.ds(sid * _RTB + k * _ZB, _ZB)])

    @pl.when(sid == _NS - 1)
    def _():
        pltpu.sync_copy(rb0.at[pl.ds(0, _TAIL)],
                        acc_sh.at[pl.ds(_NS * _RTB, _TAIL)])

    plsc.subcore_barrier()

    def _wait_gather(c, j):
        pltpu.make_async_copy(m_hbm.at[src_v.at[c]], bufs[j], gsems[j]).wait()

    def _wait_scatter(j):
        pltpu.make_async_copy(bufs[j], acc_sh.at[dst_v.at[0]], ssems[j]).wait()

    def block(b, carry):
        pltpu.sync_copy(src_hbm.at[wid, b], src_v)
        pltpu.sync_copy(dst_hbm.at[wid, b], dst_v)
        pltpu.async_copy(m_hbm.at[src_v.at[0]], bufs[0], gsems[0])
        pltpu.async_copy(m_hbm.at[src_v.at[1]], bufs[1], gsems[1])

        def pair(i, carry2):
            c0 = 2 * i
            _wait_gather(c0, 0)
            pltpu.sync_copy(bufs[0], acc_sh.at[dst_v.at[c0]], add=True)

            @pl.when(i < _NPAIR - 1)
            def _():
                pltpu.async_copy(m_hbm.at[src_v.at[c0 + 2]], bufs[0],
                                 gsems[0])
            _wait_gather(c0 + 1, 1)
            pltpu.sync_copy(bufs[1], acc_sh.at[dst_v.at[c0 + 1]], add=True)

            @pl.when(i < _NPAIR - 1)
            def _():
                pltpu.async_copy(m_hbm.at[src_v.at[c0 + 3]], bufs[1],
                                 gsems[1])
            return carry2
        return lax.fori_loop(0, _NPAIR, pair, carry)
    lax.fori_loop(0, _NB, block, 0)

    plsc.subcore_barrier()
    pltpu.sync_copy(acc_sh.at[pl.ds(sid * _RTB, _RTB)],
                    out_hbm.at[cid, pl.ds(sid * _RTB, _RTB)])

    @pl.when(sid == _NS - 1)
    def _():
        pltpu.sync_copy(acc_sh.at[pl.ds(_NS * _RTB, _TAIL)],
                        out_hbm.at[cid, pl.ds(_NS * _RTB, _TAIL)])


_agg_call = pl.kernel(
    _agg_body,
    out_type=jax.ShapeDtypeStruct((_NC, _N, _OUT), jnp.float32),
    mesh=plsc.VectorSubcoreMesh(core_axis_name="c", subcore_axis_name="s"),
    scratch_types=(
        [pltpu.VMEM((_CB, _CH), jnp.int32)] * 2
        + [pltpu.VMEM((_CH, _OUT), jnp.float32)] * 2
        + [pltpu.SemaphoreType.DMA] * 4
        + [pltpu.VMEM_SHARED((_N, _OUT), jnp.float32)]
    ),
)


def kernel(x, edge_index, graph_ids, labels, W_lin, b_lin, W_ih, W_hh,
           b_ih, b_hh, W_gate, b_gate, W_out, b_out):
    feat = jnp.concatenate(
        [x, jnp.zeros((_N, _OUT - _ANN), jnp.float32)], axis=1)
    src_rs = edge_index[0].reshape(_NW, _NB, _CB, _CH)
    dst_rs = edge_index[1].reshape(_NW, _NB, _CB, _CH)
    b_lin2 = b_lin[None, :]
    b_ih2 = b_ih[None, :]
    b_hh2 = b_hh[None, :]
    b_out2 = b_out[None, :]
    gid3 = graph_ids.reshape(_N // _RB, 1, _RB)
    lab2 = labels[None, :]

    h = feat
    m = _edge_lin(feat, W_lin, b_lin2)
    for _ in range(_STEPS):
        p_parts = _agg_call(m, src_rs, dst_rs)
        h, m = _gru_step(p_parts, h, W_lin, b_lin2, W_ih, W_hh, b_ih2, b_hh2)
    loss2, preds2 = _readout(h, x, gid3, lab2, W_out, b_out2)
    return (loss2.reshape(()), preds2.reshape(_GRAPHS))
